# baseline (speedup 1.0000x reference)
import jax
import jax.numpy as jnp
from jax.experimental import pallas as pl

_P, _C = 262144, 21


def _prep_body(logits_ref, out_ref):
    x = logits_ref[...]
    m = jnp.max(x, axis=1, keepdims=True)
    lse = m + jnp.log(jnp.sum(jnp.exp(x - m), axis=1, keepdims=True))
    out_ref[...] = x - lse


def kernel(logits, labels):
    logp = pl.pallas_call(
        _prep_body,
        out_shape=jax.ShapeDtypeStruct((_P, _C), jnp.float32),
        grid=(64,),
        in_specs=[pl.BlockSpec((_P // 64, _C), lambda i: (i, 0))],
        out_specs=pl.BlockSpec((_P // 64, _C), lambda i: (i, 0)),
    )(logits)
    classes = jnp.arange(_C)
    fg = (labels[:, None] == classes[None, :]).astype(jnp.float32)
    errors = jnp.abs(fg - logp)
    order = jnp.argsort(-errors, axis=0)
    errors_sorted = jnp.take_along_axis(errors, order, axis=0)
    fg_sorted = jnp.take_along_axis(fg, order, axis=0)
    gts = jnp.sum(fg_sorted, axis=0)
    intersection = gts[None, :] - jnp.cumsum(fg_sorted, axis=0)
    union = gts[None, :] + jnp.cumsum(1.0 - fg_sorted, axis=0)
    jaccard = 1.0 - intersection / union
    grad = jnp.concatenate([jaccard[:1], jaccard[1:] - jaccard[:-1]], axis=0)
    loss_per_class = jnp.sum(errors_sorted * grad, axis=0)
    present = (gts > 0).astype(jnp.float32)
    return jnp.sum(loss_per_class * present) / jnp.maximum(jnp.sum(present), 1.0)


# speedup vs baseline: 3.9980x; 3.9980x over previous
import jax
import jax.numpy as jnp
from jax.experimental import pallas as pl
from jax.experimental.pallas import tpu as pltpu

_P, _C = 262144, 21
_R, _L = 2048, 128
_CH = 16


def _iota(shape, dim):
    return jax.lax.broadcasted_iota(jnp.int32, shape, dim)


def _keys_body(xt_ref, lab_ref, out_ref):
    x = xt_ref[...]
    lab = lab_ref[...]
    m = jnp.max(x, axis=0)
    lse = m + jnp.log(jnp.sum(jnp.exp(x - m[None]), axis=0))
    for c in range(_C):
        fg = lab == c
        e = lse - x[c] + fg.astype(jnp.float32)
        bits = jax.lax.bitcast_convert_type(e, jnp.int32)
        out_ref[c] = jnp.bitwise_or(
            jnp.bitwise_and(bits, -2), fg.astype(jnp.int32))


def _ce_sub(X, s, dirm):
    rb = jnp.bitwise_and(_iota(X.shape, 0), s) != 0
    Pv = jnp.where(rb, pltpu.roll(X, s, 0), pltpu.roll(X, X.shape[0] - s, 0))
    take_max = dirm == jnp.logical_not(rb)
    return jnp.where(take_max, jnp.maximum(X, Pv), jnp.minimum(X, Pv))


def _ce_lane(X, s, dirm):
    lb = jnp.bitwise_and(_iota(X.shape, 1), s) != 0
    Pv = jnp.where(lb, pltpu.roll(X, s, 1), pltpu.roll(X, X.shape[1] - s, 1))
    take_max = dirm == jnp.logical_not(lb)
    return jnp.where(take_max, jnp.maximum(X, Pv), jnp.minimum(X, Pv))


def _sort_body(in_ref, out_ref, S):
    S[...] = in_ref[0]
    riota = _iota((128, _L), 0)
    liota = _iota((128, _L), 1)

    def p0_chunk(c, _):
        X = S[pl.ds(c * 128, 128), :]
        for k in range(1, 8):
            if k < 7:
                dirm = jnp.bitwise_and(riota, 1 << k) == 0
            else:
                dirm = jnp.bitwise_and(c, 1) == 0
            for j in range(k - 1, -1, -1):
                X = _ce_sub(X, 1 << j, dirm)
        S[pl.ds(c * 128, 128), :] = X
        return 0

    jax.lax.fori_loop(0, _CH, p0_chunk, 0, unroll=False)

    for k in range(8, 19):
        size = 1 << k

        if k >= 12:
            def lane_chunk(c, _, k=k):
                X = S[pl.ds(c * 128, 128), :]
                dirm = jnp.bitwise_and(liota, 1 << (k - 11)) == 0
                for j in range(k - 1, 10, -1):
                    X = _ce_lane(X, 1 << (j - 11), dirm)
                S[pl.ds(c * 128, 128), :] = X
                return 0

            jax.lax.fori_loop(0, _CH, lane_chunk, 0, unroll=False)

        jmax = min(k - 1, 10)
        if jmax >= 7:
            def b1_j(a, _, k=k, size=size, jmax=jmax):
                j = jmax - a
                mm = jnp.left_shift(jnp.int32(1), j - 7)

                def b1_q(q, _):
                    lo = jnp.bitwise_or(
                        jnp.left_shift(jnp.bitwise_and(q, -mm), 1),
                        jnp.bitwise_and(q, mm - 1))
                    A = S[pl.ds(lo * 128, 128), :]
                    B = S[pl.ds((lo + mm) * 128, 128), :]
                    i_low = liota * _R + lo * 128 + riota
                    dirm = jnp.bitwise_and(i_low, size) == 0
                    mn = jnp.minimum(A, B)
                    mx = jnp.maximum(A, B)
                    S[pl.ds(lo * 128, 128), :] = jnp.where(dirm, mx, mn)
                    S[pl.ds((lo + mm) * 128, 128), :] = jnp.where(dirm, mn, mx)
                    return 0

                jax.lax.fori_loop(0, 8, b1_q, 0, unroll=False)
                return 0

            jax.lax.fori_loop(0, jmax - 6, b1_j, 0, unroll=False)

        def b2_chunk(c, _, size=size):
            X = S[pl.ds(c * 128, 128), :]
            i0 = liota * _R + c * 128 + riota
            dirm = jnp.bitwise_and(i0, size) == 0
            for j in range(6, -1, -1):
                X = _ce_sub(X, 1 << j, dirm)
            S[pl.ds(c * 128, 128), :] = X
            return 0

        jax.lax.fori_loop(0, _CH, b2_chunk, 0, unroll=False)

    out_ref[0] = S[...]


def _loss_body(in_ref, loss_ref, g_ref):
    def cs(t, acc):
        u = in_ref[0, pl.ds(t * 256, 256), :]
        return acc + jnp.sum(
            jnp.bitwise_and(u, 1).astype(jnp.float32), axis=0, keepdims=True)

    colsum = jax.lax.fori_loop(0, 8, cs, jnp.zeros((1, _L), jnp.float32))
    G = jnp.sum(colsum)
    Mstrict = (_iota((_L, _L), 0) < _iota((_L, _L), 1)).astype(jnp.float32)
    lane_excl = jnp.dot(colsum, Mstrict, preferred_element_type=jnp.float32)
    T = (_iota((256, 256), 0) >= _iota((256, 256), 1)).astype(jnp.float32)

    u0 = in_ref[0, 0:1, :]
    e0 = jax.lax.bitcast_convert_type(u0, jnp.float32)
    head_last = jnp.where(_iota((1, _L), 1) < _L - 1,
                          pltpu.roll(e0, _L - 1, 1), 0.0)
    rio = _iota((256, _L), 0)
    lio = _iota((256, _L), 1)

    def chunk(t, carry):
        acc, rowc = carry
        u = in_ref[0, pl.ds(t * 256, 256), :]
        fg = jnp.bitwise_and(u, 1).astype(jnp.float32)
        e = jax.lax.bitcast_convert_type(u, jnp.float32)
        F = jnp.dot(T, fg, preferred_element_type=jnp.float32) + rowc + lane_excl
        i1 = (lio * _R + t * 256 + rio + 1).astype(jnp.float32)
        jac = 1.0 - (G - F) / (G + i1 - F)
        un = in_ref[0, pl.ds(jnp.minimum(t * 256 + 256, _R - 1), 1), :]
        e_nh = jax.lax.bitcast_convert_type(un, jnp.float32)
        e_nh = jnp.where(t == 7, head_last, e_nh)
        e_next = jnp.concatenate([e[1:], e_nh], axis=0)
        acc = acc + jnp.sum(jac * (e - e_next))
        rowc_new = rowc + jnp.sum(fg, axis=0, keepdims=True)
        return acc, rowc_new

    acc, _ = jax.lax.fori_loop(
        0, 8, chunk, (jnp.float32(0.0), jnp.zeros((1, _L), jnp.float32)))
    loss_ref[0] = jnp.broadcast_to(acc, (8, _L))
    g_ref[0] = jnp.broadcast_to(G, (8, _L))


def kernel(logits, labels):
    xt = jnp.swapaxes(logits, 0, 1).reshape(_C, _R, _L)
    lab3 = labels.astype(jnp.int32).reshape(_R, _L)

    keys = pl.pallas_call(
        _keys_body,
        grid=(8,),
        in_specs=[
            pl.BlockSpec((_C, 256, _L), lambda i: (0, i, 0)),
            pl.BlockSpec((256, _L), lambda i: (i, 0)),
        ],
        out_specs=pl.BlockSpec((_C, 256, _L), lambda i: (0, i, 0)),
        out_shape=jax.ShapeDtypeStruct((_C, _R, _L), jnp.int32),
    )(xt, lab3)

    skeys = pl.pallas_call(
        _sort_body,
        grid=(_C,),
        in_specs=[pl.BlockSpec((1, _R, _L), lambda c: (c, 0, 0))],
        out_specs=pl.BlockSpec((1, _R, _L), lambda c: (c, 0, 0)),
        out_shape=jax.ShapeDtypeStruct((_C, _R, _L), jnp.int32),
        scratch_shapes=[pltpu.VMEM((_R, _L), jnp.int32)],
    )(keys)

    loss_pc, g_pc = pl.pallas_call(
        _loss_body,
        grid=(_C,),
        in_specs=[pl.BlockSpec((1, _R, _L), lambda c: (c, 0, 0))],
        out_specs=[
            pl.BlockSpec((1, 8, _L), lambda c: (c, 0, 0)),
            pl.BlockSpec((1, 8, _L), lambda c: (c, 0, 0)),
        ],
        out_shape=[
            jax.ShapeDtypeStruct((_C, 8, _L), jnp.float32),
            jax.ShapeDtypeStruct((_C, 8, _L), jnp.float32),
        ],
    )(skeys)

    lpc = loss_pc[:, 0, 0]
    present = (g_pc[:, 0, 0] > 0).astype(jnp.float32)
    return jnp.sum(lpc * present) / jnp.maximum(jnp.sum(present), 1.0)
